# Initial kernel scaffold; baseline (speedup 1.0000x reference)
import jax
import jax.numpy as jnp
from jax.experimental import pallas as pl


def kernel(x, edge_index, edge_weight, W_lin, W_root, b_root):
    raise NotImplementedError("write your pallas kernel here")


# speedup vs baseline: 2.7222x; 2.7222x over previous
import functools

import jax
import jax.numpy as jnp
from jax import lax
from jax.experimental import pallas as pl
from jax.experimental.pallas import tpu as pltpu
from jax.experimental.pallas import tpu_sc as plsc

N = 10000
IN_CH = 256
OUT_CH = 256
HALF = OUT_CH // 2

NC = 2
NS = 16
LANES = 16

CHUNK = 128
NP = 10112
ROWS_PER_SUB = NP // NS


def _prep_body(x_ref, wl_ref, wr_ref, b_ref, y0_ref, y1_ref, root_ref):
    xb = x_ref[...]
    y = jnp.dot(xb, wl_ref[...], preferred_element_type=jnp.float32)
    y0_ref[...] = y[:, :HALF]
    y1_ref[...] = y[:, HALF:]
    root_ref[...] = (
        jnp.dot(xb, wr_ref[...], preferred_element_type=jnp.float32) + b_ref[...]
    )


def _tc_prep(x, W_lin, W_root, b_root2d):
    nb = 1000
    return pl.pallas_call(
        _prep_body,
        grid=(N // nb,),
        in_specs=[
            pl.BlockSpec((nb, IN_CH), lambda i: (i, 0)),
            pl.BlockSpec((IN_CH, OUT_CH), lambda i: (0, 0)),
            pl.BlockSpec((IN_CH, OUT_CH), lambda i: (0, 0)),
            pl.BlockSpec((1, OUT_CH), lambda i: (0, 0)),
        ],
        out_specs=[
            pl.BlockSpec((nb, HALF), lambda i: (i, 0)),
            pl.BlockSpec((nb, HALF), lambda i: (i, 0)),
            pl.BlockSpec((nb, OUT_CH), lambda i: (i, 0)),
        ],
        out_shape=[
            jax.ShapeDtypeStruct((N, HALF), jnp.float32),
            jax.ShapeDtypeStruct((N, HALF), jnp.float32),
            jax.ShapeDtypeStruct((N, OUT_CH), jnp.float32),
        ],
    )(x, W_lin, W_root, b_root2d)


def _sc_edge(y0, y1, src, dst, ew, n_chunks):
    mesh = plsc.VectorSubcoreMesh(
        core_axis_name="c", subcore_axis_name="s", num_cores=NC, num_subcores=NS
    )
    out_type = (
        jax.ShapeDtypeStruct((NP, HALF), jnp.float32),
        jax.ShapeDtypeStruct((NP, HALF), jnp.float32),
        jax.ShapeDtypeStruct((CHUNK, HALF), jnp.float32),
    )
    scratch_types = [
        pltpu.VMEM((CHUNK,), jnp.int32),
        pltpu.VMEM((CHUNK,), jnp.int32),
        pltpu.VMEM((CHUNK,), jnp.int32),
        pltpu.VMEM((CHUNK,), jnp.float32),
        pltpu.VMEM((CHUNK, HALF), jnp.float32),
        pltpu.VMEM((CHUNK, HALF), jnp.float32),
        pltpu.VMEM_SHARED((NP, HALF), jnp.float32),
        pltpu.VMEM_SHARED((CHUNK, HALF), jnp.float32),
    ]

    per_sub = n_chunks * CHUNK

    @functools.partial(
        pl.kernel, out_type=out_type, mesh=mesh, scratch_types=scratch_types
    )
    def run(y0_h, y1_h, src_h, dst_h, ew_h, acc0_h, acc1_h, deg_h,
            src_v, dst_v, idx_v, ew_v, rows_v, degl_v, acc_s, deg_s):
        c = lax.axis_index("c")
        s = lax.axis_index("s")
        zero = jnp.zeros((LANES,), jnp.float32)
        iota = lax.iota(jnp.int32, LANES)

        def zb(i, _):
            for k in range(HALF // LANES):
                rows_v[i, pl.ds(k * LANES, LANES)] = zero
                degl_v[i, pl.ds(k * LANES, LANES)] = zero
            return _
        lax.fori_loop(0, CHUNK, zb, 0)

        def fill_idx(base):
            for k in range(CHUNK // LANES):
                idx_v[pl.ds(k * LANES, LANES)] = base + k * LANES + iota

        nbase = s * ROWS_PER_SUB
        n_node_chunks = ROWS_PER_SUB // CHUNK
        for t in range(n_node_chunks):
            fill_idx(nbase + t * CHUNK)
            pltpu.sync_copy(rows_v, acc_s.at[idx_v])
        rem = ROWS_PER_SUB - n_node_chunks * CHUNK
        if rem:
            fill_idx(nbase + ROWS_PER_SUB - CHUNK)
            pltpu.sync_copy(rows_v, acc_s.at[idx_v])

        @pl.when(c + s == 0)
        def _():
            fill_idx(0)
            pltpu.sync_copy(rows_v, deg_s.at[idx_v])
        plsc.subcore_barrier()

        ebase = s * per_sub

        def chunk_body(t, carry):
            base = ebase + t * CHUNK
            pltpu.sync_copy(src_h.at[pl.ds(base, CHUNK)], src_v)
            pltpu.sync_copy(dst_h.at[pl.ds(base, CHUNK)], dst_v)
            pltpu.sync_copy(ew_h.at[pl.ds(base, CHUNK)], ew_v)

            @pl.when(c == 0)
            def _():
                pltpu.sync_copy(y0_h.at[src_v], rows_v)

            @pl.when(c == 1)
            def _():
                pltpu.sync_copy(y1_h.at[src_v], rows_v)

            def group_body(g, gcarry):
                ew16 = ew_v[pl.ds(g * LANES, LANES)]
                dst16 = dst_v[pl.ds(g * LANES, LANES)]
                for j in range(LANES):
                    e = g * LANES + j
                    w = jnp.full((LANES,), ew16[j], jnp.float32)
                    for k in range(HALF // LANES):
                        sl = pl.ds(k * LANES, LANES)
                        rows_v[e, sl] = rows_v[e, sl] * w
                    dj = dst16[j]
                    drow = dj // HALF
                    dcol = dj - drow * HALF
                    lane = dcol % LANES
                    sld = pl.ds(dcol - lane, LANES)
                    degl_v[drow, sld] = degl_v[drow, sld] + jnp.where(
                        iota == lane, ew16[j], 0.0
                    )
                return gcarry
            lax.fori_loop(0, CHUNK // LANES, group_body, 0)

            pltpu.sync_copy(rows_v, acc_s.at[dst_v], add=True)
            return carry
        lax.fori_loop(0, n_chunks, chunk_body, 0)

        fill_idx(0)
        pltpu.sync_copy(degl_v, deg_s.at[idx_v], add=True)

        plsc.subcore_barrier()

        def wb_chunk(row0, hoff):
            fill_idx(row0)
            pltpu.sync_copy(acc_s.at[idx_v], rows_v)
            r = pl.ds(hoff, CHUNK)

            @pl.when(c == 0)
            def _():
                pltpu.sync_copy(rows_v, acc0_h.at[r])

            @pl.when(c == 1)
            def _():
                pltpu.sync_copy(rows_v, acc1_h.at[r])

        for t in range(n_node_chunks):
            wb_chunk(nbase + t * CHUNK, nbase + t * CHUNK)
        if rem:
            wb_chunk(nbase + ROWS_PER_SUB - CHUNK, nbase + ROWS_PER_SUB - CHUNK)

        @pl.when(c + s == 0)
        def _():
            fill_idx(0)
            pltpu.sync_copy(deg_s.at[idx_v], degl_v)
            pltpu.sync_copy(degl_v, deg_h)

    return run(y0, y1, src, dst, ew)


def _norm_body(a0_ref, a1_ref, deg_ref, root_ref, out_ref):
    inv = 1.0 / jnp.maximum(deg_ref[:, 0:1], 1.0)
    out_ref[:, :HALF] = a0_ref[...] * inv + root_ref[:, :HALF]
    out_ref[:, HALF:] = a1_ref[...] * inv + root_ref[:, HALF:]


def _tc_norm(acc0, acc1, deg, root):
    nb = 400
    return pl.pallas_call(
        _norm_body,
        grid=(N // nb,),
        in_specs=[
            pl.BlockSpec((nb, HALF), lambda i: (i, 0)),
            pl.BlockSpec((nb, HALF), lambda i: (i, 0)),
            pl.BlockSpec((nb, LANES), lambda i: (i, 0)),
            pl.BlockSpec((nb, OUT_CH), lambda i: (i, 0)),
        ],
        out_specs=pl.BlockSpec((nb, OUT_CH), lambda i: (i, 0)),
        out_shape=jax.ShapeDtypeStruct((N, OUT_CH), jnp.float32),
    )(acc0, acc1, deg, root)


def kernel(x, edge_index, edge_weight, W_lin, W_root, b_root):
    E = edge_weight.shape[0]
    src = edge_index[0].astype(jnp.int32)
    dst = edge_index[1].astype(jnp.int32)
    ew = edge_weight.astype(jnp.float32)

    per_sub = -(-(-(-E // NS)) // CHUNK) * CHUNK
    ep = per_sub * NS
    if ep > E:
        pad = ep - E
        src = jnp.concatenate([src, jnp.zeros((pad,), jnp.int32)])
        dst = jnp.concatenate([dst, jnp.zeros((pad,), jnp.int32)])
        ew = jnp.concatenate([ew, jnp.zeros((pad,), jnp.float32)])

    y0, y1, root = _tc_prep(x, W_lin, W_root, b_root.reshape(1, OUT_CH))
    acc0, acc1, degp = _sc_edge(y0, y1, src, dst, ew, per_sub // CHUNK)
    deg = jnp.broadcast_to(degp.reshape(-1)[:NP, None], (NP, LANES))
    return _tc_norm(acc0, acc1, deg, root)


# speedup vs baseline: 3.0076x; 1.1049x over previous
import functools

import jax
import jax.numpy as jnp
from jax import lax
from jax.experimental import pallas as pl
from jax.experimental.pallas import tpu as pltpu
from jax.experimental.pallas import tpu_sc as plsc

N = 10000
IN_CH = 256
OUT_CH = 256
HALF = OUT_CH // 2

NC = 2
NS = 16
LANES = 16

CHUNK = 128
NP = 10112
ROWS_PER_SUB = NP // NS
DROWS = 80


def _prep_body(x_ref, wl_ref, wr_ref, b_ref, y0_ref, y1_ref, root_ref):
    xb = x_ref[...]
    y = jnp.dot(xb, wl_ref[...], preferred_element_type=jnp.float32)
    y0_ref[...] = y[:, :HALF]
    y1_ref[...] = y[:, HALF:]
    root_ref[...] = (
        jnp.dot(xb, wr_ref[...], preferred_element_type=jnp.float32) + b_ref[...]
    )


def _tc_prep(x, W_lin, W_root, b_root2d):
    nb = 1000
    return pl.pallas_call(
        _prep_body,
        grid=(N // nb,),
        in_specs=[
            pl.BlockSpec((nb, IN_CH), lambda i: (i, 0)),
            pl.BlockSpec((IN_CH, OUT_CH), lambda i: (0, 0)),
            pl.BlockSpec((IN_CH, OUT_CH), lambda i: (0, 0)),
            pl.BlockSpec((1, OUT_CH), lambda i: (0, 0)),
        ],
        out_specs=[
            pl.BlockSpec((nb, HALF), lambda i: (i, 0)),
            pl.BlockSpec((nb, HALF), lambda i: (i, 0)),
            pl.BlockSpec((nb, OUT_CH), lambda i: (i, 0)),
        ],
        out_shape=[
            jax.ShapeDtypeStruct((N, HALF), jnp.float32),
            jax.ShapeDtypeStruct((N, HALF), jnp.float32),
            jax.ShapeDtypeStruct((N, OUT_CH), jnp.float32),
        ],
    )(x, W_lin, W_root, b_root2d)


def _sc_edge(y0, y1, src, dst, ew, n_chunks):
    mesh = plsc.VectorSubcoreMesh(
        core_axis_name="c", subcore_axis_name="s", num_cores=NC, num_subcores=NS
    )
    out_type = (
        jax.ShapeDtypeStruct((NP, HALF), jnp.float32),
        jax.ShapeDtypeStruct((NP, HALF), jnp.float32),
        jax.ShapeDtypeStruct((DROWS, HALF), jnp.float32),
    )
    per_sub = n_chunks * CHUNK

    scratch_types = [
        pltpu.VMEM((CHUNK,), jnp.int32),
        pltpu.VMEM((CHUNK,), jnp.int32),
        pltpu.VMEM((CHUNK,), jnp.int32),
        pltpu.VMEM((CHUNK,), jnp.int32),
        pltpu.VMEM((CHUNK,), jnp.float32),
        pltpu.VMEM((CHUNK,), jnp.float32),
        pltpu.VMEM((CHUNK,), jnp.int32),
        pltpu.VMEM((DROWS,), jnp.int32),
        pltpu.VMEM((CHUNK, HALF), jnp.float32),
        pltpu.VMEM((CHUNK, HALF), jnp.float32),
        pltpu.VMEM((DROWS, HALF), jnp.float32),
        pltpu.VMEM_SHARED((NP, HALF), jnp.float32),
        pltpu.VMEM_SHARED((DROWS, HALF), jnp.float32),
        pltpu.SemaphoreType.DMA,
        pltpu.SemaphoreType.DMA,
        pltpu.SemaphoreType.DMA,
        pltpu.SemaphoreType.DMA,
    ]

    @functools.partial(
        pl.kernel, out_type=out_type, mesh=mesh, scratch_types=scratch_types
    )
    def run(y0_h, y1_h, src_h, dst_h, ew_h, acc0_h, acc1_h, deg_h,
            si0, si1, di0, di1, ev0, ev1, idx_v, degi_v,
            r0, r1, degl_v, acc_s, deg_s, gs0, gs1, ss0, ss1):
        c = lax.axis_index("c")
        s = lax.axis_index("s")
        zero = jnp.zeros((LANES,), jnp.float32)
        iota = lax.iota(jnp.int32, LANES)
        SRCV = (si0, si1)
        DSTV = (di0, di1)
        EWV = (ev0, ev1)
        ROWS = (r0, r1)
        GS = (gs0, gs1)
        SS = (ss0, ss1)

        def zb(i, _):
            for k in range(HALF // LANES):
                r0[i, pl.ds(k * LANES, LANES)] = zero
            return _
        lax.fori_loop(0, CHUNK, zb, 0)

        def zd(i, _):
            for k in range(HALF // LANES):
                degl_v[i, pl.ds(k * LANES, LANES)] = zero
            return _
        lax.fori_loop(0, DROWS, zd, 0)

        def fill_idx(base):
            for k in range(CHUNK // LANES):
                idx_v[pl.ds(k * LANES, LANES)] = base + k * LANES + iota

        for k in range(DROWS // LANES):
            degi_v[pl.ds(k * LANES, LANES)] = k * LANES + iota

        nbase = s * ROWS_PER_SUB
        n_node_chunks = ROWS_PER_SUB // CHUNK
        for t in range(n_node_chunks):
            fill_idx(nbase + t * CHUNK)
            pltpu.sync_copy(r0, acc_s.at[idx_v])
        rem = ROWS_PER_SUB - n_node_chunks * CHUNK
        if rem:
            fill_idx(nbase + ROWS_PER_SUB - CHUNK)
            pltpu.sync_copy(r0, acc_s.at[idx_v])

        @pl.when(c + s == 0)
        def _():
            pltpu.sync_copy(degl_v, deg_s.at[degi_v])
        plsc.subcore_barrier()


        ebase = s * per_sub

        def issue_gather(slot, t):
            e0 = ebase + t * CHUNK
            pltpu.sync_copy(src_h.at[pl.ds(e0, CHUNK)], SRCV[slot])
            pltpu.sync_copy(dst_h.at[pl.ds(e0, CHUNK)], DSTV[slot])
            pltpu.sync_copy(ew_h.at[pl.ds(e0, CHUNK)], EWV[slot])

            @pl.when(c == 0)
            def _():
                pltpu.async_copy(y0_h.at[SRCV[slot]], ROWS[slot], GS[slot])

            @pl.when(c == 1)
            def _():
                pltpu.async_copy(y1_h.at[SRCV[slot]], ROWS[slot], GS[slot])

        def wait_gather(slot):
            pltpu.make_async_copy(y0_h.at[SRCV[slot]], ROWS[slot], GS[slot]).wait()

        def issue_scatter(slot):
            pltpu.async_copy(ROWS[slot], acc_s.at[DSTV[slot]], SS[slot], add=True)

        def wait_scatter(slot):
            pltpu.make_async_copy(ROWS[slot], acc_s.at[DSTV[slot]], SS[slot]).wait()

        def scale(slot):
            rv = ROWS[slot]

            def group_body(g, gcarry):
                off = g * LANES
                ew16 = EWV[slot][pl.ds(off, LANES)]
                dst16 = DSTV[slot][pl.ds(off, LANES)]
                for j in range(LANES):
                    e = g * LANES + j
                    w = jnp.full((LANES,), ew16[j], jnp.float32)
                    for k in range(HALF // LANES):
                        sl = pl.ds(k * LANES, LANES)
                        rv[e, sl] = rv[e, sl] * w
                    dj = dst16[j]
                    drow = dj // HALF
                    dcol = dj - drow * HALF
                    lane = dcol % LANES
                    sld = pl.ds(dcol - lane, LANES)
                    degl_v[drow, sld] = degl_v[drow, sld] + jnp.where(
                        iota == lane, ew16[j], 0.0
                    )
                return gcarry
            lax.fori_loop(0, CHUNK // LANES, group_body, 0)

        def step(t, cur, nxt):
            @pl.when(t >= 1)
            def _():
                wait_scatter(nxt)

            @pl.when(t + 1 < n_chunks)
            def _():
                issue_gather(nxt, t + 1)

            wait_gather(cur)
            scale(cur)
            issue_scatter(cur)

        issue_gather(0, 0)

        def pair_body(jj, carry):
            t0 = jj * 2
            step(t0, 0, 1)
            step(t0 + 1, 1, 0)
            return carry
        lax.fori_loop(0, n_chunks // 2, pair_body, 0)

        wait_scatter((n_chunks - 1) % 2)

        pltpu.sync_copy(degl_v, deg_s.at[degi_v], add=True)

        plsc.subcore_barrier()

        def wb_chunk(row0, hoff):
            fill_idx(row0)
            pltpu.sync_copy(acc_s.at[idx_v], r0)
            r = pl.ds(hoff, CHUNK)

            @pl.when(c == 0)
            def _():
                pltpu.sync_copy(r0, acc0_h.at[r])

            @pl.when(c == 1)
            def _():
                pltpu.sync_copy(r0, acc1_h.at[r])

        for t in range(n_node_chunks):
            wb_chunk(nbase + t * CHUNK, nbase + t * CHUNK)
        if rem:
            wb_chunk(nbase + ROWS_PER_SUB - CHUNK, nbase + ROWS_PER_SUB - CHUNK)

        @pl.when(c + s == 0)
        def _():
            pltpu.sync_copy(deg_s.at[degi_v], degl_v)
            pltpu.sync_copy(degl_v, deg_h)

    return run(y0, y1, src, dst, ew)


def _norm_body(a0_ref, a1_ref, deg_ref, root_ref, out_ref):
    inv = 1.0 / jnp.maximum(deg_ref[:, 0:1], 1.0)
    out_ref[:, :HALF] = a0_ref[...] * inv + root_ref[:, :HALF]
    out_ref[:, HALF:] = a1_ref[...] * inv + root_ref[:, HALF:]


def _tc_norm(acc0, acc1, deg, root):
    nb = 400
    return pl.pallas_call(
        _norm_body,
        grid=(N // nb,),
        in_specs=[
            pl.BlockSpec((nb, HALF), lambda i: (i, 0)),
            pl.BlockSpec((nb, HALF), lambda i: (i, 0)),
            pl.BlockSpec((nb, LANES), lambda i: (i, 0)),
            pl.BlockSpec((nb, OUT_CH), lambda i: (i, 0)),
        ],
        out_specs=pl.BlockSpec((nb, OUT_CH), lambda i: (i, 0)),
        out_shape=jax.ShapeDtypeStruct((N, OUT_CH), jnp.float32),
    )(acc0, acc1, deg, root)


def kernel(x, edge_index, edge_weight, W_lin, W_root, b_root):
    E = edge_weight.shape[0]
    src = edge_index[0].astype(jnp.int32)
    dst = edge_index[1].astype(jnp.int32)
    ew = edge_weight.astype(jnp.float32)

    n_ch = -(-(-(-E // NS)) // CHUNK)
    n_ch = -(-n_ch // 2) * 2
    per_sub = n_ch * CHUNK
    ep = per_sub * NS
    if ep > E:
        pad = ep - E
        src = jnp.concatenate([src, jnp.zeros((pad,), jnp.int32)])
        dst = jnp.concatenate([dst, jnp.zeros((pad,), jnp.int32)])
        ew = jnp.concatenate([ew, jnp.zeros((pad,), jnp.float32)])

    y0, y1, root = _tc_prep(x, W_lin, W_root, b_root.reshape(1, OUT_CH))
    acc0, acc1, degp = _sc_edge(y0, y1, src, dst, ew, per_sub // CHUNK)
    deg = jnp.broadcast_to(degp.reshape(-1)[:NP, None], (NP, LANES))
    return _tc_norm(acc0, acc1, deg, root)
